# baseline (speedup 1.0000x reference)
import jax
import jax.numpy as jnp
from jax import lax
from jax.experimental import pallas as pl
from jax.experimental.pallas import tpu as pltpu
from jax.experimental.pallas import tpu_sc as plsc

BATCH = 16384
INP_DIM = 39
N_CONT = 13
N_CAT = 26
VOCAB = 100000
EMB_DIM = 32
EMB_TOT = N_CAT * EMB_DIM
OUT_DIM = N_CONT + EMB_TOT
SLAB = 4 * EMB_DIM
N_SLABS = N_CAT * VOCAB // 4

NC, NS, L = 2, 16, 16
NW = NC * NS
ROWS_PER_W = BATCH // NW
CHUNK = 128
N_CHUNKS = ROWS_PER_W // CHUNK
GROUPS = CHUNK // L


def _sc_body(x_hbm, tab_hbm, emb_hbm, xv, idx4, subb, stage4, stout, gsem, wsem):
    wid = lax.axis_index("s") * NC + lax.axis_index("c")
    iota = lax.iota(jnp.int32, L)

    def chunk_body(g, _):
        base = wid * ROWS_PER_W + g * CHUNK
        pltpu.sync_copy(x_hbm.at[pl.ds(base * INP_DIM, CHUNK * INP_DIM)], xv)

        def col_body(c, _):
            p = jax.lax.rem(c, 2)
            q = 1 - p

            @pl.when(c < N_CAT)
            def _fire():
                for k in range(GROUPS):
                    flat = (iota + (k * L)) * INP_DIM + (N_CONT + c)
                    gv = plsc.load_gather(xv, [flat]) + c * VOCAB
                    idx4[p, 0, pl.ds(k * L, L)] = lax.shift_right_logical(gv, 2)
                    subb[p, pl.ds(k * L, L)] = lax.shift_left(
                        lax.bitwise_and(gv, 3), 5
                    )
                pltpu.make_async_copy(
                    tab_hbm.at[idx4.at[p, 0]], stage4.at[p], gsem.at[p]
                ).start()

            @pl.when(c >= 1)
            def _drain():
                cm1 = c - 1
                pltpu.make_async_copy(
                    tab_hbm.at[idx4.at[q, 0]], stage4.at[q], gsem.at[q]
                ).wait()

                @pl.when(c >= 3)
                def _wb_drain():
                    pltpu.make_async_copy(
                        stout.at[q],
                        emb_hbm.at[c - 3, pl.ds(base, CHUNK), :],
                        wsem.at[q],
                    ).wait()

                def grp_body(k, _):
                    rows = iota + k * L
                    subv = subb[q, pl.ds(k * L, L)]
                    for pos in range(EMB_DIM):
                        lanes = subv + pos
                        v = plsc.load_gather(stage4.at[q], [rows, lanes])
                        plsc.store_scatter(
                            stout.at[q],
                            [rows, jnp.broadcast_to(pos, (L,))],
                            v,
                        )
                    return _
                lax.fori_loop(0, GROUPS, grp_body, None)
                pltpu.make_async_copy(
                    stout.at[q],
                    emb_hbm.at[cm1, pl.ds(base, CHUNK), :],
                    wsem.at[q],
                ).start()
            return _

        lax.fori_loop(0, N_CAT + 1, col_body, None)

        for cc in (N_CAT - 2, N_CAT - 1):
            pltpu.make_async_copy(
                stout.at[cc % 2],
                emb_hbm.at[cc, pl.ds(base, CHUNK), :],
                wsem.at[cc % 2],
            ).wait()
        return _

    lax.fori_loop(0, N_CHUNKS, chunk_body, None)


TC_ROWS = 512


def _tc_body(x_ref, emb_ref, out_ref):
    cont = x_ref[:, :N_CONT].astype(jnp.float32)
    cols = [cont] + [emb_ref[c] for c in range(N_CAT)]
    out_ref[...] = jnp.concatenate(cols, axis=1)


@jax.jit
def _run(x, xflat, tab):
    mesh = plsc.VectorSubcoreMesh(
        core_axis_name="c", subcore_axis_name="s", num_cores=NC, num_subcores=NS
    )
    emb = pl.kernel(
        _sc_body,
        out_type=jax.ShapeDtypeStruct((N_CAT, BATCH, EMB_DIM), jnp.float32),
        mesh=mesh,
        compiler_params=pltpu.CompilerParams(needs_layout_passes=False),
        scratch_types=[
            pltpu.VMEM((CHUNK * INP_DIM,), jnp.int32),
            pltpu.VMEM((2, 1, CHUNK), jnp.int32),
            pltpu.VMEM((2, CHUNK), jnp.int32),
            pltpu.VMEM((2, CHUNK, SLAB), jnp.float32),
            pltpu.VMEM((2, CHUNK, EMB_DIM), jnp.float32),
            pltpu.SemaphoreType.DMA((2,)),
            pltpu.SemaphoreType.DMA((2,)),
        ],
    )(xflat, tab)

    return pl.pallas_call(
        _tc_body,
        grid=(BATCH // TC_ROWS,),
        in_specs=[
            pl.BlockSpec((TC_ROWS, INP_DIM), lambda i: (i, 0)),
            pl.BlockSpec((N_CAT, TC_ROWS, EMB_DIM), lambda i: (0, i, 0)),
        ],
        out_specs=pl.BlockSpec((TC_ROWS, OUT_DIM), lambda i: (i, 0)),
        out_shape=jax.ShapeDtypeStruct((BATCH, OUT_DIM), jnp.float32),
    )(x, emb)


def kernel(x, tables):
    x = x.astype(jnp.int32)
    tab = tables.reshape(N_SLABS, SLAB)
    return _run(x, x.reshape(-1), tab)


# speedup vs baseline: 4.9822x; 4.9822x over previous
import jax
import jax.numpy as jnp
from jax import lax
from jax.experimental import pallas as pl
from jax.experimental.pallas import tpu as pltpu
from jax.experimental.pallas import tpu_sc as plsc

BATCH = 16384
INP_DIM = 39
N_CONT = 13
N_CAT = 26
VOCAB = 100000
EMB_DIM = 32
EMB_TOT = N_CAT * EMB_DIM
OUT_DIM = N_CONT + EMB_TOT

NC, NS, L = 2, 16, 16
NW = NC * NS
ROWS_PER_W = EMB_TOT // NW
HALF = BATCH // 2
KH = HALF // L


def _sc_body(xt_hbm, t2_hbm, out_hbm, rowb, idxb, outb, sem):
    wid = lax.axis_index("s") * NC + lax.axis_index("c")

    @pl.when(wid < N_CONT)
    def _cont():
        pltpu.sync_copy(xt_hbm.at[wid, :], idxb)
        for h in range(2):
            def conv_body(k, _):
                v = idxb[pl.ds(h * HALF + k * L, L)]
                outb[pl.ds(k * L, L)] = v.astype(jnp.float32)
                return _
            lax.fori_loop(0, KH, conv_body, None)
            pltpu.sync_copy(outb, out_hbm.at[wid, pl.ds(h * HALF, HALF)])

    def row_body(i, c_prev):
        r = wid * ROWS_PER_W + i
        c = lax.div(r, EMB_DIM)

        @pl.when(c != c_prev)
        def _load_idx():
            pltpu.sync_copy(xt_hbm.at[N_CONT + c, :], idxb)

        pltpu.sync_copy(t2_hbm.at[r, :], rowb)
        for h in range(2):
            def gat_body(k, _):
                iv = idxb[pl.ds(h * HALF + k * L, L)]
                outb[pl.ds(k * L, L)] = plsc.load_gather(rowb, [iv])
                return _
            lax.fori_loop(0, KH, gat_body, None)
            pltpu.sync_copy(
                outb, out_hbm.at[N_CONT + r, pl.ds(h * HALF, HALF)]
            )
        return c

    lax.fori_loop(0, ROWS_PER_W, row_body, jnp.int32(-1))


TC_COLS = 512


def _tc_body(t_ref, out_ref):
    out_ref[...] = t_ref[...].T


@jax.jit
def _run(xt, t2):
    mesh = plsc.VectorSubcoreMesh(
        core_axis_name="c", subcore_axis_name="s", num_cores=NC, num_subcores=NS
    )
    out_t = pl.kernel(
        _sc_body,
        out_type=jax.ShapeDtypeStruct((OUT_DIM, BATCH), jnp.float32),
        mesh=mesh,
        compiler_params=pltpu.CompilerParams(needs_layout_passes=False),
        scratch_types=[
            pltpu.VMEM((VOCAB,), jnp.float32),
            pltpu.VMEM((BATCH,), jnp.int32),
            pltpu.VMEM((HALF,), jnp.float32),
            pltpu.SemaphoreType.DMA,
        ],
    )(xt, t2)

    return out_t.T


def kernel(x, tables):
    xt = x.astype(jnp.int32).T
    t2 = tables.transpose(0, 2, 1).reshape(EMB_TOT, VOCAB)
    return _run(xt, t2)
